# Initial kernel scaffold; baseline (speedup 1.0000x reference)
import jax
import jax.numpy as jnp
from jax.experimental import pallas as pl


def kernel(X, nucleo_emb, pos_emb):
    raise NotImplementedError("write your pallas kernel here")


# speedup vs baseline: 5.8043x; 5.8043x over previous
import functools

import jax
import jax.numpy as jnp
from jax import lax
from jax.experimental import pallas as pl
from jax.experimental.pallas import tpu as pltpu
from jax.experimental.pallas import tpu_sc as plsc

BATCH = 4096
SEQ = 200
NNUC = 4
DIM = 128

NW = 32
ROWS = BATCH * SEQ
RPW = ROWS // NW
CH = 256
NCH = RPW // CH
NG = CH // 128
LANES = 16


def _comb_body(nuc_ref, pos_ref, out_ref):
    pos = pos_ref[...]
    for n in range(NNUC):
        out_ref[:, n, :] = pos + nuc_ref[n, :]


def _build_comb(nucleo_emb, pos_emb):
    comb = pl.pallas_call(
        _comb_body,
        out_shape=jax.ShapeDtypeStruct((SEQ, NNUC, DIM), jnp.float32),
    )(nucleo_emb, pos_emb)
    return comb.reshape(SEQ * NNUC, DIM)


def _sc_body(x_hbm, comb_hbm, out_hbm, x_v, idx_v, rows_v, sem):
    wid = lax.axis_index("s") * 2 + lax.axis_index("c")
    base0 = wid * RPW
    lane = lax.iota(jnp.int32, LANES)

    def chunk(k, carry):
        base = base0 + k * CH
        pltpu.sync_copy(x_hbm.at[pl.ds(base, CH)], x_v)
        for g in range(CH // LANES):
            r = (base + g * LANES) + lane
            idx = (r % SEQ) * 4 + x_v[pl.ds(g * LANES, LANES)]
            idx_v[g * LANES // 128, pl.ds((g * LANES) % 128, LANES)] = idx
        copies = []
        for g in range(NG):
            copies.append(
                pltpu.async_copy(
                    comb_hbm.at[idx_v.at[g]],
                    rows_v.at[pl.ds(g * 128, 128)],
                    sem,
                )
            )
        for c in copies:
            c.wait()
        pltpu.sync_copy(rows_v, out_hbm.at[pl.ds(base, CH)])
        return carry

    lax.fori_loop(0, NCH, chunk, 0)


def kernel(X, nucleo_emb, pos_emb):
    comb = _build_comb(nucleo_emb, pos_emb)
    x_flat = X.reshape(ROWS)

    mesh = plsc.VectorSubcoreMesh(core_axis_name="c", subcore_axis_name="s")
    sc_embed = functools.partial(
        pl.kernel,
        mesh=mesh,
        out_type=jax.ShapeDtypeStruct((ROWS, DIM), jnp.float32),
        scratch_types=[
            pltpu.VMEM((CH,), jnp.int32),
            pltpu.VMEM((NG, 128), jnp.int32),
            pltpu.VMEM((CH, DIM), jnp.float32),
            pltpu.SemaphoreType.DMA,
        ],
    )(_sc_body)

    out = sc_embed(x_flat, comb)
    return out.reshape(BATCH, SEQ, DIM)


# speedup vs baseline: 15.3897x; 2.6514x over previous
import functools

import jax
import jax.numpy as jnp
from jax import lax
from jax.experimental import pallas as pl
from jax.experimental.pallas import tpu as pltpu
from jax.experimental.pallas import tpu_sc as plsc

BATCH = 4096
SEQ = 200
NNUC = 4
DIM = 128

NW = 32
ROWS = BATCH * SEQ
RPW = ROWS // NW
UNIT = 128
NUNITS = RPW // UNIT
NRING = 4
LANES = 16


def _comb_body(nuc_ref, pos_ref, out_ref):
    pos = pos_ref[...]
    for n in range(NNUC):
        out_ref[:, n, :] = pos + nuc_ref[n, :]


def _build_comb(nucleo_emb, pos_emb):
    comb = pl.pallas_call(
        _comb_body,
        out_shape=jax.ShapeDtypeStruct((SEQ, NNUC, DIM), jnp.float32),
    )(nucleo_emb, pos_emb)
    return comb.reshape(SEQ * NNUC, DIM)


def _sc_body(x_hbm, comb_hbm, out_hbm, comb_sh, x_v,
             r0, r1, r2, r3, sg0, sg1, sg2, sg3, ss0, ss1, ss2, ss3):
    rings = (r0, r1, r2, r3)
    gsems = (sg0, sg1, sg2, sg3)
    ssems = (ss0, ss1, ss2, ss3)

    wid = lax.axis_index("s") * 2 + lax.axis_index("c")
    base0 = wid * RPW
    lane = lax.iota(jnp.int32, LANES)

    pltpu.sync_copy(x_hbm.at[pl.ds(base0, RPW)], x_v)

    @pl.when(lax.axis_index("s") == 0)
    def _stage():
        pltpu.sync_copy(comb_hbm, comb_sh)

    plsc.subcore_barrier()

    def compute_idx(k):
        for g in range(UNIT // LANES):
            off = k * UNIT + g * LANES
            r = (base0 + off) + lane
            x_v[pl.ds(off, LANES)] = (r % SEQ) * 4 + x_v[pl.ds(off, LANES)]

    def start_gather(k, u):
        compute_idx(k)
        pltpu.async_copy(
            comb_sh.at[x_v.at[pl.ds(k * UNIT, UNIT)]], rings[u], gsems[u]
        )

    def start_scatter(k, u):
        pltpu.async_copy(
            rings[u], out_hbm.at[pl.ds(base0 + k * UNIT, UNIT)], ssems[u]
        )

    def wait_gather(u):
        pltpu.make_async_copy(comb_hbm.at[pl.ds(0, UNIT)], rings[u],
                              gsems[u]).wait()

    def wait_scatter(u):
        pltpu.make_async_copy(rings[u], out_hbm.at[pl.ds(0, UNIT)],
                              ssems[u]).wait()

    for u in range(NRING):
        start_gather(u, u)

    def body(j, carry):
        k = j * NRING
        for u in range(NRING):
            wait_gather(u)
            start_scatter(k + u, u)
        for u in range(NRING):
            wait_scatter(u)
            start_gather(k + NRING + u, u)
        return carry

    lax.fori_loop(0, NUNITS // NRING - 1, body, 0)

    for u in range(NRING):
        wait_gather(u)
        start_scatter(NUNITS - NRING + u, u)
    for u in range(NRING):
        wait_scatter(u)


def kernel(X, nucleo_emb, pos_emb):
    comb = _build_comb(nucleo_emb, pos_emb)
    x_flat = X.reshape(ROWS)

    mesh = plsc.VectorSubcoreMesh(core_axis_name="c", subcore_axis_name="s")
    sc_embed = functools.partial(
        pl.kernel,
        mesh=mesh,
        out_type=jax.ShapeDtypeStruct((ROWS, DIM), jnp.float32),
        scratch_types=[
            pltpu.VMEM_SHARED((SEQ * NNUC, DIM), jnp.float32),
            pltpu.VMEM((RPW,), jnp.int32),
            pltpu.VMEM((UNIT, DIM), jnp.float32),
            pltpu.VMEM((UNIT, DIM), jnp.float32),
            pltpu.VMEM((UNIT, DIM), jnp.float32),
            pltpu.VMEM((UNIT, DIM), jnp.float32),
            pltpu.SemaphoreType.DMA,
            pltpu.SemaphoreType.DMA,
            pltpu.SemaphoreType.DMA,
            pltpu.SemaphoreType.DMA,
            pltpu.SemaphoreType.DMA,
            pltpu.SemaphoreType.DMA,
            pltpu.SemaphoreType.DMA,
            pltpu.SemaphoreType.DMA,
        ],
    )(_sc_body)

    out = sc_embed(x_flat, comb)
    return out.reshape(BATCH, SEQ, DIM)
